# baseline (speedup 1.0000x reference)
import functools

import jax
import jax.numpy as jnp
from jax import lax
from jax.experimental import pallas as pl
from jax.experimental.pallas import tpu as pltpu
from jax.experimental.pallas import tpu_sc as plsc

NUM_RGAP = 128
NUM_SGAP = 128
EMB = 128
TABLE_ROWS = NUM_RGAP + NUM_SGAP + 128

_info = plsc.get_sparse_core_info()
_NC, _NS = _info.num_cores, _info.num_subcores
_NW = _NC * _NS

CHUNK = 256


def _tg_kernel(n_total, rg_hbm, sg_hbm, pc_hbm, w_hbm, out_hbm,
               w_v, rg0, sg0, pc0, rg1, sg1, pc1, out0, out1,
               si0, si1, so0, so1):
    wid = lax.axis_index("s") * _NC + lax.axis_index("c")
    per_w = n_total // _NW
    base = wid * per_w
    n_pairs = per_w // (2 * CHUNK)

    idx_bufs = ((rg0, sg0, pc0), (rg1, sg1, pc1))
    out_bufs = (out0, out1)
    si = (si0, si1)
    so = (so0, so1)

    pltpu.sync_copy(w_hbm, w_v)

    def start_idx(ci, slot):
        g = base + ci * CHUNK
        rgb, sgb, pcb = idx_bufs[slot]
        pltpu.async_copy(rg_hbm.at[pl.ds(g, CHUNK)], rgb, si[slot])
        pltpu.async_copy(sg_hbm.at[pl.ds(g, CHUNK)], sgb, si[slot])
        pltpu.async_copy(pc_hbm.at[pl.ds(g, CHUNK)], pcb, si[slot])

    def wait_idx(slot):
        rgb, sgb, pcb = idx_bufs[slot]
        pltpu.make_async_copy(rg_hbm.at[pl.ds(base, CHUNK)], rgb, si[slot]).wait()
        pltpu.make_async_copy(sg_hbm.at[pl.ds(base, CHUNK)], sgb, si[slot]).wait()
        pltpu.make_async_copy(pc_hbm.at[pl.ds(base, CHUNK)], pcb, si[slot]).wait()

    def start_out(ci, slot):
        g = base + ci * CHUNK
        pltpu.async_copy(out_bufs[slot], out_hbm.at[pl.ds(g, CHUNK)], so[slot])

    def wait_out(slot):
        pltpu.make_async_copy(out_bufs[slot],
                              out_hbm.at[pl.ds(base, CHUNK)], so[slot]).wait()

    def compute(slot):
        nd = EMB // 32
        rgb, sgb, pcb = idx_bufs[slot]
        ob = out_bufs[slot]

        def emit_compute(e, a, b, c):
            alo, blo, clo = [], [], []
            for d in range(nd):
                alo.append(plsc.bitcast(a[d] << 16, jnp.float32))
                yield
                blo.append(plsc.bitcast(b[d] << 16, jnp.float32))
                yield
                clo.append(plsc.bitcast(c[d] << 16, jnp.float32))
                yield
            for d in range(nd):
                s = alo[d] + blo[d]
                yield
                s = s + clo[d]
                yield
                ob[e, pl.ds(d * 32, 16)] = s
                yield
                h = plsc.bitcast(a[d], jnp.float32) + plsc.bitcast(b[d], jnp.float32)
                yield
                h = h + plsc.bitcast(c[d], jnp.float32)
                yield
                ob[e, pl.ds(d * 32 + 16, 16)] = h
                yield

        def grp_body(g, _):
            e0 = g * 32
            rgv = [rgb[pl.ds(e0 + 16 * h, 16)] for h in range(2)]
            sgv = [sgb[pl.ds(e0 + 16 * h, 16)] + NUM_RGAP for h in range(2)]
            pcv = [pcb[pl.ds(e0 + 16 * h, 16)] + (NUM_RGAP + NUM_SGAP)
                   for h in range(2)]
            def extract_offs(k):
                h, kk = divmod(k, 16)
                return (rgv[h][kk] * (EMB // 2),
                        sgv[h][kk] * (EMB // 2),
                        pcv[h][kk] * (EMB // 2))

            offq = [extract_offs(0), extract_offs(1)]
            pendings = []
            for k in range(32):
                if k + 2 < 32:
                    offq.append(extract_offs(k + 2))
                o1, o2, o3 = offq[k]
                offs = [(o1, 0), (o2, 1), (o3, 2)]
                loads = [[], [], []]
                for d in range(nd):
                    for o, t in offs:
                        loads[t].append(w_v[pl.ds(o + d * 16, 16)])
                        if len(pendings) == 2:
                            for _ in range(3):
                                next(pendings[0], None)
                if len(pendings) == 2:
                    pendings.pop(0)
                pendings.append(emit_compute(e0 + k, *loads))
            for gen in pendings:
                for _ in gen:
                    pass
            return ()

        @plsc.parallel_loop(0, CHUNK // 32, unroll=2)
        def _(g):
            grp_body(g, ())

    start_idx(0, 0)
    start_idx(1, 1)

    def pair_body(p, _):
        for slot in range(2):
            ci = 2 * p + slot

            @pl.when(p >= 1)
            def _():
                wait_out(slot)

            wait_idx(slot)
            compute(slot)
            start_out(ci, slot)

            @pl.when(p + 1 < n_pairs)
            def _():
                start_idx(ci + 2, slot)
        return ()

    lax.fori_loop(0, n_pairs, pair_body, ())
    wait_out(0)
    wait_out(1)


def kernel(rgap, sgap, pcount, W):
    B, S = rgap.shape
    n = B * S
    rg = rgap.reshape(n).astype(jnp.int32)
    sg = sgap.reshape(n).astype(jnp.int32)
    pc = pcount.reshape(n).astype(jnp.int32)
    w = jax.lax.bitcast_convert_type(
        W.astype(jnp.bfloat16)
         .reshape(TABLE_ROWS, EMB // 32, 2, 16)
         .swapaxes(2, 3),
        jnp.uint32).reshape(TABLE_ROWS * EMB // 2)

    mesh = plsc.VectorSubcoreMesh(core_axis_name="c", subcore_axis_name="s")
    run = pl.kernel(
        functools.partial(_tg_kernel, n),
        out_type=jax.ShapeDtypeStruct((n, EMB), jnp.float32),
        mesh=mesh,
        compiler_params=pltpu.CompilerParams(needs_layout_passes=False),
        scratch_types=[
            pltpu.VMEM((TABLE_ROWS * EMB // 2,), jnp.uint32),
            pltpu.VMEM((CHUNK,), jnp.int32),
            pltpu.VMEM((CHUNK,), jnp.int32),
            pltpu.VMEM((CHUNK,), jnp.int32),
            pltpu.VMEM((CHUNK,), jnp.int32),
            pltpu.VMEM((CHUNK,), jnp.int32),
            pltpu.VMEM((CHUNK,), jnp.int32),
            pltpu.VMEM((CHUNK, EMB), jnp.float32),
            pltpu.VMEM((CHUNK, EMB), jnp.float32),
            pltpu.SemaphoreType.DMA,
            pltpu.SemaphoreType.DMA,
            pltpu.SemaphoreType.DMA,
            pltpu.SemaphoreType.DMA,
        ],
    )
    out = run(rg, sg, pc, w)
    return out.reshape(B, S, EMB)


# speedup vs baseline: 1.1166x; 1.1166x over previous
import functools

import jax
import jax.numpy as jnp
from jax import lax
from jax.experimental import pallas as pl
from jax.experimental.pallas import tpu as pltpu
from jax.experimental.pallas import tpu_sc as plsc

NUM_RGAP = 128
NUM_SGAP = 128
EMB = 128
TABLE_ROWS = NUM_RGAP + NUM_SGAP + 128

_info = plsc.get_sparse_core_info()
_NC, _NS = _info.num_cores, _info.num_subcores
_NW = _NC * _NS

CHUNK = 256


def _tg_kernel(n_total, rg_hbm, sg_hbm, pc_hbm, w_hbm, out_hbm,
               w_v, rg0, sg0, pc0, rg1, sg1, pc1, out0, out1,
               si0, si1, so0, so1):
    wid = lax.axis_index("s") * _NC + lax.axis_index("c")
    per_w = n_total // _NW
    base = wid * per_w
    n_pairs = per_w // (2 * CHUNK)

    idx_bufs = ((rg0, sg0, pc0), (rg1, sg1, pc1))
    out_bufs = (out0, out1)
    si = (si0, si1)
    so = (so0, so1)

    pltpu.sync_copy(w_hbm, w_v)

    def start_idx(ci, slot):
        g = base + ci * CHUNK
        rgb, sgb, pcb = idx_bufs[slot]
        pltpu.async_copy(rg_hbm.at[pl.ds(g, CHUNK)], rgb, si[slot])
        pltpu.async_copy(sg_hbm.at[pl.ds(g, CHUNK)], sgb, si[slot])
        pltpu.async_copy(pc_hbm.at[pl.ds(g, CHUNK)], pcb, si[slot])

    def wait_idx(slot):
        rgb, sgb, pcb = idx_bufs[slot]
        pltpu.make_async_copy(rg_hbm.at[pl.ds(base, CHUNK)], rgb, si[slot]).wait()
        pltpu.make_async_copy(sg_hbm.at[pl.ds(base, CHUNK)], sgb, si[slot]).wait()
        pltpu.make_async_copy(pc_hbm.at[pl.ds(base, CHUNK)], pcb, si[slot]).wait()

    def start_out(ci, slot):
        g = base + ci * CHUNK
        pltpu.async_copy(out_bufs[slot], out_hbm.at[pl.ds(g, CHUNK)], so[slot])

    def wait_out(slot):
        pltpu.make_async_copy(out_bufs[slot],
                              out_hbm.at[pl.ds(base, CHUNK)], so[slot]).wait()

    def compute(slot):
        nd = EMB // 32
        rgb, sgb, pcb = idx_bufs[slot]
        ob = out_bufs[slot]

        def emit_compute(e, a, b, c):
            alo, blo, clo = [], [], []
            for d in range(nd):
                alo.append(plsc.bitcast(a[d] << 16, jnp.float32))
                yield
                blo.append(plsc.bitcast(b[d] << 16, jnp.float32))
                yield
                clo.append(plsc.bitcast(c[d] << 16, jnp.float32))
                yield
            for d in range(nd):
                s = alo[d] + blo[d]
                yield
                s = s + clo[d]
                yield
                ob[e, pl.ds(d * 32, 16)] = s
                yield
                h = plsc.bitcast(a[d], jnp.float32) + plsc.bitcast(b[d], jnp.float32)
                yield
                h = h + plsc.bitcast(c[d], jnp.float32)
                yield
                ob[e, pl.ds(d * 32 + 16, 16)] = h
                yield

        def grp_body(g, _):
            e0 = g * 32
            rgv = [rgb[pl.ds(e0 + 16 * h, 16)] for h in range(2)]
            sgv = [sgb[pl.ds(e0 + 16 * h, 16)] + NUM_RGAP for h in range(2)]
            pcv = [pcb[pl.ds(e0 + 16 * h, 16)] + (NUM_RGAP + NUM_SGAP)
                   for h in range(2)]
            def extract_offs(k):
                h, kk = divmod(k, 16)
                return (rgv[h][kk] * (EMB // 2),
                        sgv[h][kk] * (EMB // 2),
                        pcv[h][kk] * (EMB // 2))

            offq = [extract_offs(0), extract_offs(1)]
            pendings = []
            for k in range(32):
                if k + 2 < 32:
                    offq.append(extract_offs(k + 2))
                o1, o2, o3 = offq[k]
                offs = [(o1, 0), (o2, 1), (o3, 2)]
                loads = [[], [], []]
                for d in range(nd):
                    for o, t in offs:
                        loads[t].append(w_v[pl.ds(o + d * 16, 16)])
                        if len(pendings) == 2:
                            for _ in range(3):
                                next(pendings[0], None)
                if len(pendings) == 2:
                    pendings.pop(0)
                pendings.append(emit_compute(e0 + k, *loads))
            for gen in pendings:
                for _ in gen:
                    pass
            return ()

        lax.fori_loop(0, CHUNK // 32, grp_body, ())

    start_idx(0, 0)
    start_idx(1, 1)

    def pair_body(p, _):
        for slot in range(2):
            ci = 2 * p + slot

            @pl.when(p >= 1)
            def _():
                wait_out(slot)

            wait_idx(slot)
            compute(slot)
            start_out(ci, slot)

            @pl.when(p + 1 < n_pairs)
            def _():
                start_idx(ci + 2, slot)
        return ()

    lax.fori_loop(0, n_pairs, pair_body, ())
    wait_out(0)
    wait_out(1)


def kernel(rgap, sgap, pcount, W):
    B, S = rgap.shape
    n = B * S
    rg = rgap.reshape(n).astype(jnp.int32)
    sg = sgap.reshape(n).astype(jnp.int32)
    pc = pcount.reshape(n).astype(jnp.int32)
    w = jax.lax.bitcast_convert_type(
        W.astype(jnp.bfloat16)
         .reshape(TABLE_ROWS, EMB // 32, 2, 16)
         .swapaxes(2, 3),
        jnp.uint32).reshape(TABLE_ROWS * EMB // 2)

    mesh = plsc.VectorSubcoreMesh(core_axis_name="c", subcore_axis_name="s")
    run = pl.kernel(
        functools.partial(_tg_kernel, n),
        out_type=jax.ShapeDtypeStruct((n, EMB), jnp.float32),
        mesh=mesh,
        compiler_params=pltpu.CompilerParams(needs_layout_passes=False),
        scratch_types=[
            pltpu.VMEM((TABLE_ROWS * EMB // 2,), jnp.uint32),
            pltpu.VMEM((CHUNK,), jnp.int32),
            pltpu.VMEM((CHUNK,), jnp.int32),
            pltpu.VMEM((CHUNK,), jnp.int32),
            pltpu.VMEM((CHUNK,), jnp.int32),
            pltpu.VMEM((CHUNK,), jnp.int32),
            pltpu.VMEM((CHUNK,), jnp.int32),
            pltpu.VMEM((CHUNK, EMB), jnp.float32),
            pltpu.VMEM((CHUNK, EMB), jnp.float32),
            pltpu.SemaphoreType.DMA,
            pltpu.SemaphoreType.DMA,
            pltpu.SemaphoreType.DMA,
            pltpu.SemaphoreType.DMA,
        ],
    )
    out = run(rg, sg, pc, w)
    return out.reshape(B, S, EMB)


# speedup vs baseline: 1.1178x; 1.0012x over previous
import functools

import jax
import jax.numpy as jnp
from jax import lax
from jax.experimental import pallas as pl
from jax.experimental.pallas import tpu as pltpu
from jax.experimental.pallas import tpu_sc as plsc

NUM_RGAP = 128
NUM_SGAP = 128
EMB = 128
TABLE_ROWS = NUM_RGAP + NUM_SGAP + 128

_info = plsc.get_sparse_core_info()
_NC, _NS = _info.num_cores, _info.num_subcores
_NW = _NC * _NS

CHUNK = 256


def _tg_kernel(n_total, rg_hbm, sg_hbm, pc_hbm, w_hbm, out_hbm,
               w_v, rg0, sg0, pc0, rg1, sg1, pc1, out0, out1,
               si0, si1, so0, so1):
    wid = lax.axis_index("s") * _NC + lax.axis_index("c")
    per_w = n_total // _NW
    base = wid * per_w
    n_pairs = per_w // (2 * CHUNK)

    idx_bufs = ((rg0, sg0, pc0), (rg1, sg1, pc1))
    out_bufs = (out0, out1)
    si = (si0, si1)
    so = (so0, so1)

    pltpu.sync_copy(w_hbm, w_v)

    def start_idx(ci, slot):
        g = base + ci * CHUNK
        rgb, sgb, pcb = idx_bufs[slot]
        pltpu.async_copy(rg_hbm.at[pl.ds(g, CHUNK)], rgb, si[slot])
        pltpu.async_copy(sg_hbm.at[pl.ds(g, CHUNK)], sgb, si[slot])
        pltpu.async_copy(pc_hbm.at[pl.ds(g, CHUNK)], pcb, si[slot])

    def wait_idx(slot):
        rgb, sgb, pcb = idx_bufs[slot]
        pltpu.make_async_copy(rg_hbm.at[pl.ds(base, CHUNK)], rgb, si[slot]).wait()
        pltpu.make_async_copy(sg_hbm.at[pl.ds(base, CHUNK)], sgb, si[slot]).wait()
        pltpu.make_async_copy(pc_hbm.at[pl.ds(base, CHUNK)], pcb, si[slot]).wait()

    def start_out(ci, slot):
        g = base + ci * CHUNK
        pltpu.async_copy(out_bufs[slot], out_hbm.at[pl.ds(g, CHUNK)], so[slot])

    def wait_out(slot):
        pltpu.make_async_copy(out_bufs[slot],
                              out_hbm.at[pl.ds(base, CHUNK)], so[slot]).wait()

    def compute(slot):
        nd = EMB // 32
        rgb, sgb, pcb = idx_bufs[slot]
        ob = out_bufs[slot]

        def emit_compute(e, a, b, c):
            alo, blo, clo = [], [], []
            for d in range(nd):
                alo.append(plsc.bitcast(a[d] << 16, jnp.float32))
                yield
                blo.append(plsc.bitcast(b[d] << 16, jnp.float32))
                yield
                clo.append(plsc.bitcast(c[d] << 16, jnp.float32))
                yield
            for d in range(nd):
                s = alo[d] + blo[d]
                yield
                s = s + clo[d]
                yield
                ob[e, pl.ds(d * 32, 16)] = s
                yield
                h = plsc.bitcast(a[d], jnp.float32) + plsc.bitcast(b[d], jnp.float32)
                yield
                h = h + plsc.bitcast(c[d], jnp.float32)
                yield
                ob[e, pl.ds(d * 32 + 16, 16)] = h
                yield

        def grp_body(g, _):
            e0 = g * 32
            rgv = [rgb[pl.ds(e0 + 16 * h, 16)] for h in range(2)]
            sgv = [sgb[pl.ds(e0 + 16 * h, 16)] + NUM_RGAP for h in range(2)]
            pcv = [pcb[pl.ds(e0 + 16 * h, 16)] + (NUM_RGAP + NUM_SGAP)
                   for h in range(2)]
            def extract_offs(k):
                h, kk = divmod(k, 16)
                return (rgv[h][kk] * (EMB // 2),
                        sgv[h][kk] * (EMB // 2),
                        pcv[h][kk] * (EMB // 2))

            offq = [extract_offs(0), extract_offs(1)]
            pendings = []
            for k in range(32):
                if k + 2 < 32:
                    offq.append(extract_offs(k + 2))
                o1, o2, o3 = offq[k]
                offs = [(o1, 0), (o2, 1), (o3, 2)]
                loads = [[], [], []]
                for d in range(nd):
                    for o, t in offs:
                        loads[t].append(w_v[pl.ds(o + d * 16, 16)])
                        if len(pendings) == 2:
                            for _ in range(3):
                                next(pendings[0], None)
                if len(pendings) == 2:
                    pendings.pop(0)
                pendings.append(emit_compute(e0 + k, *loads))
            for gen in pendings:
                for _ in gen:
                    pass
            return ()

        @plsc.parallel_loop(0, CHUNK // 32)
        def _(g):
            grp_body(g, ())

    start_idx(0, 0)
    start_idx(1, 1)

    def pair_body(p, _):
        for slot in range(2):
            ci = 2 * p + slot

            @pl.when(p >= 1)
            def _():
                wait_out(slot)

            wait_idx(slot)
            compute(slot)
            start_out(ci, slot)

            @pl.when(p + 1 < n_pairs)
            def _():
                start_idx(ci + 2, slot)
        return ()

    lax.fori_loop(0, n_pairs, pair_body, ())
    wait_out(0)
    wait_out(1)


def kernel(rgap, sgap, pcount, W):
    B, S = rgap.shape
    n = B * S
    rg = rgap.reshape(n).astype(jnp.int32)
    sg = sgap.reshape(n).astype(jnp.int32)
    pc = pcount.reshape(n).astype(jnp.int32)
    w = jax.lax.bitcast_convert_type(
        W.astype(jnp.bfloat16)
         .reshape(TABLE_ROWS, EMB // 32, 2, 16)
         .swapaxes(2, 3),
        jnp.uint32).reshape(TABLE_ROWS * EMB // 2)

    mesh = plsc.VectorSubcoreMesh(core_axis_name="c", subcore_axis_name="s")
    run = pl.kernel(
        functools.partial(_tg_kernel, n),
        out_type=jax.ShapeDtypeStruct((n, EMB), jnp.float32),
        mesh=mesh,
        compiler_params=pltpu.CompilerParams(needs_layout_passes=False),
        scratch_types=[
            pltpu.VMEM((TABLE_ROWS * EMB // 2,), jnp.uint32),
            pltpu.VMEM((CHUNK,), jnp.int32),
            pltpu.VMEM((CHUNK,), jnp.int32),
            pltpu.VMEM((CHUNK,), jnp.int32),
            pltpu.VMEM((CHUNK,), jnp.int32),
            pltpu.VMEM((CHUNK,), jnp.int32),
            pltpu.VMEM((CHUNK,), jnp.int32),
            pltpu.VMEM((CHUNK, EMB), jnp.float32),
            pltpu.VMEM((CHUNK, EMB), jnp.float32),
            pltpu.SemaphoreType.DMA,
            pltpu.SemaphoreType.DMA,
            pltpu.SemaphoreType.DMA,
            pltpu.SemaphoreType.DMA,
        ],
    )
    out = run(rg, sg, pc, w)
    return out.reshape(B, S, EMB)
